# baseline (speedup 1.0000x reference)
import jax
import jax.numpy as jnp
from jax.experimental import pallas as pl
from jax.experimental.pallas import tpu as pltpu

STRIDE = 8.0
GRID = 128
N = GRID * GRID
B = 4
MAX_DET = 100
CONF_TH = 0.25
IOU_TH = 0.45
WIN = 48
NG = GRID // 8


def _nms_kernel(r0, r1, r2, r3, r4, r5, ocx, ocy, ow, oh, oconf,
                cxs, cys, ws, hs, confs, scs, rmx):
    R = B * GRID
    col = jax.lax.broadcasted_iota(jnp.int32, (R, GRID), 1)
    row = jax.lax.broadcasted_iota(jnp.int32, (R, GRID), 0)
    gx = col.astype(jnp.float32)
    gy = (row % GRID).astype(jnp.float32)

    cx = (jax.nn.sigmoid(r0[...]) * 2.0 - 0.5 + gx) * STRIDE
    cy = (jax.nn.sigmoid(r1[...]) * 2.0 - 0.5 + gy) * STRIDE
    w = (jax.nn.sigmoid(r2[...]) * 2.0) ** 2 * (STRIDE * 4.0)
    h = (jax.nn.sigmoid(r3[...]) * 2.0) ** 2 * (STRIDE * 4.0)
    conf = jax.nn.sigmoid(r4[...]) * jax.nn.sigmoid(r5[...])
    sc = jnp.where(conf >= CONF_TH, conf, -1.0)

    cxs[...] = cx
    cys[...] = cy
    ws[...] = w
    hs[...] = h
    confs[...] = conf
    scs[...] = sc

    rmx[...] = jnp.concatenate(
        [jnp.max(sc[8 * k:8 * k + 8, :], axis=0, keepdims=True)
         for k in range(B * NG)], axis=0)

    zeros_out = jnp.zeros((1, 128), jnp.float32)
    for oref in (ocx, ocy, ow, oh, oconf):
        for b in range(B):
            oref[pl.ds(b, 1), :] = zeros_out

    lane_out = jax.lax.broadcasted_iota(jnp.int32, (1, 128), 1)
    iota_g = jax.lax.broadcasted_iota(jnp.int32, (NG, GRID), 0)
    blk_key = (jax.lax.broadcasted_iota(jnp.int32, (8, GRID), 0) * GRID
               + jax.lax.broadcasted_iota(jnp.int32, (8, GRID), 1))
    lane_row = jax.lax.broadcasted_iota(jnp.int32, (1, GRID), 1)
    win_row = jax.lax.broadcasted_iota(jnp.int32, (WIN, GRID), 0)
    win_col = jax.lax.broadcasted_iota(jnp.int32, (WIN, GRID), 1)

    def step(t, _):
        for b in range(B):
            base = b * GRID
            gbase = b * NG
            rm = rmx[pl.ds(gbase, NG), :]
            m = jnp.max(rm)
            valid = m > 0.0
            g = jnp.min(jnp.where(rm == m, iota_g, NG))
            g = jnp.minimum(g, NG - 1)
            blk = scs[pl.ds(base + 8 * g, 8), :]
            i = jnp.min(jnp.where(blk == m, blk_key, 8 * GRID))
            i = jnp.minimum(i, 8 * GRID - 1)
            r = 8 * g + i // GRID
            c = i % GRID

            onehot = lane_row == c

            def gather(ref):
                rowv = ref[pl.ds(base + r, 1), :]
                return jnp.sum(jnp.where(onehot, rowv, 0.0), axis=1,
                               keepdims=True)

            wcx, wcy, ww, wh, wconf = (gather(cxs), gather(cys),
                                       gather(ws), gather(hs),
                                       gather(confs))
            wx1 = wcx - ww * 0.5
            wy1 = wcy - wh * 0.5
            wx2 = wcx + ww * 0.5
            wy2 = wcy + wh * 0.5
            warea = (jnp.maximum(wx2 - wx1, 0.0)
                     * jnp.maximum(wy2 - wy1, 0.0))

            t0 = jnp.maximum(r - 17, 0)
            row0 = jnp.minimum(t0 - t0 % 8, GRID - WIN)
            wsl = pl.ds(base + row0, WIN)

            cxw = cxs[wsl, :]
            cyw = cys[wsl, :]
            www = ws[wsl, :]
            hww = hs[wsl, :]
            x1w = cxw - www * 0.5
            y1w = cyw - hww * 0.5
            x2w = cxw + www * 0.5
            y2w = cyw + hww * 0.5
            areaw = (jnp.maximum(x2w - x1w, 0.0)
                     * jnp.maximum(y2w - y1w, 0.0))
            xx1 = jnp.maximum(wx1, x1w)
            yy1 = jnp.maximum(wy1, y1w)
            xx2 = jnp.minimum(wx2, x2w)
            yy2 = jnp.minimum(wy2, y2w)
            inter = (jnp.maximum(xx2 - xx1, 0.0)
                     * jnp.maximum(yy2 - yy1, 0.0))
            iou = inter / (warea + areaw - inter + 1e-9)

            sw = scs[wsl, :]
            selfhot = (win_row == (r - row0)) & (win_col == c)
            kill = (iou > IOU_TH) | selfhot
            new_sw = jnp.where(valid & kill, -1.0, sw)
            scs[wsl, :] = new_sw

            nr = jnp.concatenate(
                [jnp.max(new_sw[8 * k:8 * k + 8, :], axis=0,
                         keepdims=True) for k in range(WIN // 8)], axis=0)
            rmx[pl.ds(gbase + row0 // 8, WIN // 8), :] = nr

            slot = (lane_out == t) & valid

            def put(ref, val):
                ref[pl.ds(b, 1), :] = jnp.where(slot, val,
                                                ref[pl.ds(b, 1), :])

            put(ocx, wcx)
            put(ocy, wcy)
            put(ow, ww)
            put(oh, wh)
            put(oconf, wconf)
        return ()

    jax.lax.fori_loop(0, MAX_DET, step, (), unroll=False)


@jax.jit
def kernel(raw):
    out_shape = [jax.ShapeDtypeStruct((B, 128), jnp.float32)] * 5
    scratch = ([pltpu.VMEM((B * GRID, GRID), jnp.float32)] * 6
               + [pltpu.VMEM((B * NG, GRID), jnp.float32)])
    raw_t = jnp.transpose(raw, (2, 0, 1)).reshape(6, B * GRID, GRID)
    ocx, ocy, ow, oh, oconf = pl.pallas_call(
        _nms_kernel,
        out_shape=out_shape,
        scratch_shapes=scratch,
    )(raw_t[0], raw_t[1], raw_t[2], raw_t[3], raw_t[4], raw_t[5])
    cls = jnp.zeros_like(oconf)
    out = jnp.stack([ocx, ocy, ow, oh, oconf, cls], axis=-1)
    return out[:, :MAX_DET, :]


# speedup vs baseline: 2.6106x; 2.6106x over previous
import jax
import jax.numpy as jnp
from jax.experimental import pallas as pl
from jax.experimental.pallas import tpu as pltpu

STRIDE = 8.0
GRID = 128
N = GRID * GRID
B = 4
MAX_DET = 100
CONF_TH = 0.25
IOU_TH = 0.45
R = 8
C = N // R


def _img_reduce(col, fn):
    per = [fn(col[8 * b:8 * b + 8, :], axis=0, keepdims=True)
           for b in range(B)]
    small = jnp.concatenate(per, axis=0)
    big = jnp.concatenate(
        [jnp.broadcast_to(p, (R, 1)) for p in per], axis=0)
    return small, big


def _nms_kernel(r0, r1, r2, r3, r4, r5, ocx, ocy, ow, oh, oconf,
                x1s, y1s, x2s, y2s, areas, cxs, cys, ws, hs, confs, scs):
    BR = B * R
    col_i = jax.lax.broadcasted_iota(jnp.int32, (BR, C), 1)
    row_i = jax.lax.broadcasted_iota(jnp.int32, (BR, C), 0)
    flat = (row_i % R) * C + col_i
    gx = (flat % GRID).astype(jnp.float32)
    gy = (flat // GRID).astype(jnp.float32)

    cx = (jax.nn.sigmoid(r0[...]) * 2.0 - 0.5 + gx) * STRIDE
    cy = (jax.nn.sigmoid(r1[...]) * 2.0 - 0.5 + gy) * STRIDE
    w = (jax.nn.sigmoid(r2[...]) * 2.0) ** 2 * (STRIDE * 4.0)
    h = (jax.nn.sigmoid(r3[...]) * 2.0) ** 2 * (STRIDE * 4.0)
    conf = jax.nn.sigmoid(r4[...]) * jax.nn.sigmoid(r5[...])

    x1 = cx - w * 0.5
    y1 = cy - h * 0.5
    x2 = cx + w * 0.5
    y2 = cy + h * 0.5
    area = jnp.maximum(x2 - x1, 0.0) * jnp.maximum(y2 - y1, 0.0)

    x1s[...] = x1
    y1s[...] = y1
    x2s[...] = x2
    y2s[...] = y2
    areas[...] = area
    cxs[...] = cx
    cys[...] = cy
    ws[...] = w
    hs[...] = h
    confs[...] = conf
    scs[...] = jnp.where(conf >= CONF_TH, conf, -1.0)

    zeros_out = jnp.zeros((B, 128), jnp.float32)
    for oref in (ocx, ocy, ow, oh, oconf):
        oref[...] = zeros_out
    lane_out = jax.lax.broadcasted_iota(jnp.int32, (B, 128), 1)

    def step(t, _):
        scores = scs[...]
        mcol = jnp.max(scores, axis=1, keepdims=True)
        m4, m = _img_reduce(mcol, jnp.max)
        valid = m > 0.0
        valid4 = m4 > 0.0
        hit = scores == m
        icol = jnp.min(jnp.where(hit, flat, N), axis=1, keepdims=True)
        _, win = _img_reduce(icol, jnp.min)
        onehot = flat == win

        def gather(ref):
            scol = jnp.sum(jnp.where(onehot, ref[...], 0.0), axis=1,
                           keepdims=True)
            return _img_reduce(scol, jnp.sum)

        (wcx4, wcx), (wcy4, wcy), (ww4, ww), (wh4, wh), (wconf4, _) = (
            gather(cxs), gather(cys), gather(ws), gather(hs), gather(confs))
        wx1 = wcx - ww * 0.5
        wy1 = wcy - wh * 0.5
        wx2 = wcx + ww * 0.5
        wy2 = wcy + wh * 0.5
        warea = (jnp.maximum(wx2 - wx1, 0.0)
                 * jnp.maximum(wy2 - wy1, 0.0))

        xx1 = jnp.maximum(wx1, x1s[...])
        yy1 = jnp.maximum(wy1, y1s[...])
        xx2 = jnp.minimum(wx2, x2s[...])
        yy2 = jnp.minimum(wy2, y2s[...])
        inter = jnp.maximum(xx2 - xx1, 0.0) * jnp.maximum(yy2 - yy1, 0.0)
        iou = inter / (warea + areas[...] - inter + 1e-9)
        kill = (iou > IOU_TH) | onehot
        scs[...] = jnp.where(valid & kill, -1.0, scores)

        slot = (lane_out == t) & valid4

        def put(ref, val4):
            ref[...] = jnp.where(slot, val4, ref[...])

        put(ocx, wcx4)
        put(ocy, wcy4)
        put(ow, ww4)
        put(oh, wh4)
        put(oconf, wconf4)
        return ()

    jax.lax.fori_loop(0, MAX_DET, step, (), unroll=False)


@jax.jit
def kernel(raw):
    out_shape = [jax.ShapeDtypeStruct((B, 128), jnp.float32)] * 5
    scratch = [pltpu.VMEM((B * R, C), jnp.float32)] * 11
    raw_t = jnp.transpose(raw, (2, 0, 1)).reshape(6, B * R, C)
    ocx, ocy, ow, oh, oconf = pl.pallas_call(
        _nms_kernel,
        out_shape=out_shape,
        scratch_shapes=scratch,
    )(raw_t[0], raw_t[1], raw_t[2], raw_t[3], raw_t[4], raw_t[5])
    cls = jnp.zeros_like(oconf)
    out = jnp.stack([ocx, ocy, ow, oh, oconf, cls], axis=-1)
    return out[:, :MAX_DET, :]
